# baseline (speedup 1.0000x reference)
import jax
import jax.numpy as jnp
from jax.experimental import pallas as pl

_N, _D = 100000, 128
_BLOCK = 25000


def _copy_body(x_ref, o_ref):
    o_ref[...] = x_ref[...]


def kernel(data_list):
    return pl.pallas_call(
        _copy_body,
        grid=(_N // _BLOCK,),
        in_specs=[pl.BlockSpec((_BLOCK, _D), lambda i: (i, 0))],
        out_specs=pl.BlockSpec((_BLOCK, _D), lambda i: (i, 0)),
        out_shape=jax.ShapeDtypeStruct((_N, _D), jnp.float32),
    )(data_list)
